# initial kernel scaffold (unmeasured)
import numpy as np
import jax
import jax.numpy as jnp
from jax import lax
from jax.experimental import pallas as pl
from jax.experimental.pallas import tpu as pltpu

N_DEV = 4
SQ = 1024
SKV = 1024
H_LOC = 8
DH = 128
DM = 1024
SCALE = 0.08838834764831843

_qb = (np.arange(SQ) // 64)[:, None]
_kb = (np.arange(SKV) // 64)[None, :]
_mask = (_qb == _kb) | (_kb == 0) | ((_qb + _kb) % 3 == 0)
_BIAS = np.where(_mask, 0.0, -1e9).astype(np.float32)


def kernel(x, Wq, K_ext, V_ext, Wo):
    my = lax.axis_index("i")
    K_sl = lax.dynamic_slice_in_dim(K_ext, my * H_LOC, H_LOC, axis=2)
    V_sl = lax.dynamic_slice_in_dim(V_ext, my * H_LOC, H_LOC, axis=2)
    K_sl = K_sl.transpose(0, 2, 1, 3).astype(jnp.bfloat16)
    V_sl = V_sl.transpose(0, 2, 1, 3).astype(jnp.bfloat16)
    x_b = x.astype(jnp.bfloat16)
    Wq_b = Wq.astype(jnp.bfloat16)
    Wo_b = Wo.astype(jnp.bfloat16)
    bias = jnp.asarray(_BIAS)

    def body(x_ref, wq_ref, k_ref, v_ref, wo_ref, bias_ref, out_ref,
             xbuf, arecv, asend, xs, xr, as_, ar):
        my_pos = lax.axis_index("i")
        left = (my_pos + N_DEV - 1) % N_DEV
        right = (my_pos + 1) % N_DEV

        def partial_for(x_val, b):
            q_all = lax.dot_general(
                x_val, wq_ref[:, :], (((1,), (0,)), ((), ())),
                preferred_element_type=jnp.float32,
            ).astype(jnp.bfloat16)
            kb = k_ref[b]
            vb = v_ref[b]
            acc = jnp.zeros((SQ, DM), jnp.float32)
            for h in range(H_LOC):
                q = q_all[:, h * DH:(h + 1) * DH]
                s = lax.dot_general(
                    q, kb[h], (((1,), (1,)), ((), ())),
                    preferred_element_type=jnp.float32,
                )
                s = s * SCALE + bias_ref[:, :]
                m = jnp.max(s, axis=1, keepdims=True)
                w = jnp.exp(s - m)
                w = w / jnp.sum(w, axis=1, keepdims=True)
                ctx = lax.dot_general(
                    w.astype(jnp.bfloat16), vb[h], (((1,), (0,)), ((), ())),
                    preferred_element_type=jnp.float32,
                )
                acc = acc + lax.dot_general(
                    ctx.astype(jnp.bfloat16), wo_ref[h * DH:(h + 1) * DH, :],
                    (((1,), (0,)), ((), ())),
                    preferred_element_type=jnp.float32,
                )
            return acc

        def xcopy(src, slot):
            return pltpu.make_async_remote_copy(
                src_ref=src, dst_ref=xbuf.at[slot],
                send_sem=xs.at[slot], recv_sem=xr.at[slot],
                device_id=(right,), device_id_type=pl.DeviceIdType.MESH,
            )

        def acopy(slot):
            return pltpu.make_async_remote_copy(
                src_ref=asend, dst_ref=arecv.at[slot],
                send_sem=as_.at[slot], recv_sem=ar.at[slot],
                device_id=(right,), device_id_type=pl.DeviceIdType.MESH,
            )

        barrier = pltpu.get_barrier_semaphore()
        for nbr in (left, right):
            pl.semaphore_signal(
                barrier, inc=1,
                device_id=(nbr,), device_id_type=pl.DeviceIdType.MESH,
            )
        pl.semaphore_wait(barrier, 2)

        cx0 = xcopy(x_ref.at[0], 0)
        cx0.start()
        cx0.wait()
        cx1 = xcopy(xbuf.at[0], 1)
        cx1.start()
        p = partial_for(xbuf[0], (my_pos + 3) % N_DEV)
        asend[:, :] = p
        ca0 = acopy(0)
        ca0.start()

        cx1.wait()
        cx2 = xcopy(xbuf.at[1], 2)
        cx2.start()
        p = partial_for(xbuf[1], (my_pos + 2) % N_DEV)
        ca0.wait()
        asend[:, :] = arecv[0] + p
        ca1 = acopy(1)
        ca1.start()

        cx2.wait()
        p = partial_for(xbuf[2], (my_pos + 1) % N_DEV)
        ca1.wait()
        asend[:, :] = arecv[1] + p
        ca2 = acopy(2)
        ca2.start()

        p = partial_for(x_ref[0], my_pos)
        ca2.wait()
        out_ref[0, :, :] = arecv[2] + p

    out = pl.pallas_call(
        body,
        out_shape=jax.ShapeDtypeStruct((1, SQ, DM), jnp.float32),
        in_specs=[pl.BlockSpec(memory_space=pltpu.VMEM)] * 6,
        out_specs=pl.BlockSpec(memory_space=pltpu.VMEM),
        scratch_shapes=[
            pltpu.VMEM((3, SQ, DM), jnp.bfloat16),
            pltpu.VMEM((3, SQ, DM), jnp.float32),
            pltpu.VMEM((SQ, DM), jnp.float32),
            pltpu.SemaphoreType.DMA((3,)),
            pltpu.SemaphoreType.DMA((3,)),
            pltpu.SemaphoreType.DMA((3,)),
            pltpu.SemaphoreType.DMA((3,)),
        ],
        compiler_params=pltpu.CompilerParams(collective_id=0),
    )(x_b, Wq_b, K_sl, V_sl, Wo_b, bias)
    return out


# baseline (device time: 259091 ns/iter reference)
import numpy as np
import jax
import jax.numpy as jnp
from jax import lax
from jax.experimental import pallas as pl
from jax.experimental.pallas import tpu as pltpu

N_DEV = 4
SQ = 1024
SKV = 1024
H_LOC = 8
DH = 128
DM = 1024
SCALE = 0.08838834764831843

_qb = (np.arange(SQ) // 64)[:, None]
_kb = (np.arange(SKV) // 64)[None, :]
_mask = (_qb == _kb) | (_kb == 0) | ((_qb + _kb) % 3 == 0)
_BIAS = np.where(_mask, 0.0, -1e9).astype(np.float32)


def kernel(x, Wq, K_ext, V_ext, Wo):
    my = lax.axis_index("i")
    K_sl = lax.dynamic_slice_in_dim(K_ext, my * H_LOC, H_LOC, axis=2)
    V_sl = lax.dynamic_slice_in_dim(V_ext, my * H_LOC, H_LOC, axis=2)
    K_sl = K_sl.transpose(0, 2, 1, 3).astype(jnp.bfloat16)
    V_sl = V_sl.transpose(0, 2, 1, 3).astype(jnp.bfloat16)
    x_b = x.astype(jnp.bfloat16)
    Wq_b = Wq.astype(jnp.bfloat16).reshape(DM, H_LOC, DH).transpose(1, 0, 2)
    Wo_b = Wo.astype(jnp.bfloat16).reshape(H_LOC, DH, DM)
    bias = jnp.asarray(_BIAS, dtype=jnp.bfloat16)

    def body(x_ref, wq_ref, k_ref, v_ref, wo_ref, bias_ref, out_ref,
             xbuf, arecv, asend, xs, xr, as_, ar):
        my_pos = lax.axis_index("i")
        left = (my_pos + N_DEV - 1) % N_DEV
        right = (my_pos + 1) % N_DEV

        def partial_for(x_val, b):

            def head_body(h, acc):
                q = lax.dot_general(
                    x_val, wq_ref[h], (((1,), (0,)), ((), ())),
                    preferred_element_type=jnp.float32,
                ).astype(jnp.bfloat16)
                s = lax.dot_general(
                    q, k_ref[b, h], (((1,), (1,)), ((), ())),
                    preferred_element_type=jnp.float32,
                )
                s = s * SCALE + bias_ref[:, :]
                m = jnp.max(s, axis=1, keepdims=True)
                w = jnp.exp(s - m)
                w = w / jnp.sum(w, axis=1, keepdims=True)
                ctx = lax.dot_general(
                    w.astype(jnp.bfloat16), v_ref[b, h],
                    (((1,), (0,)), ((), ())),
                    preferred_element_type=jnp.float32,
                )
                return acc + lax.dot_general(
                    ctx.astype(jnp.bfloat16), wo_ref[h],
                    (((1,), (0,)), ((), ())),
                    preferred_element_type=jnp.float32,
                )

            return lax.fori_loop(
                0, H_LOC, head_body, jnp.zeros((SQ, DM), jnp.float32)
            )

        def xcopy(src, slot):
            return pltpu.make_async_remote_copy(
                src_ref=src, dst_ref=xbuf.at[slot],
                send_sem=xs.at[slot], recv_sem=xr.at[slot],
                device_id=(right,), device_id_type=pl.DeviceIdType.MESH,
            )

        def acopy(slot):
            return pltpu.make_async_remote_copy(
                src_ref=asend, dst_ref=arecv.at[slot],
                send_sem=as_.at[slot], recv_sem=ar.at[slot],
                device_id=(right,), device_id_type=pl.DeviceIdType.MESH,
            )

        barrier = pltpu.get_barrier_semaphore()
        for nbr in (left, right):
            pl.semaphore_signal(
                barrier, inc=1,
                device_id=(nbr,), device_id_type=pl.DeviceIdType.MESH,
            )
        pl.semaphore_wait(barrier, 2)

        cx0 = xcopy(x_ref.at[0], 0)
        cx0.start()
        cx0.wait()
        cx1 = xcopy(xbuf.at[0], 1)
        cx1.start()
        p = partial_for(xbuf[0], (my_pos + 3) % N_DEV)
        asend[:, :] = p.astype(jnp.bfloat16)
        ca0 = acopy(0)
        ca0.start()

        cx1.wait()
        cx2 = xcopy(xbuf.at[1], 2)
        cx2.start()
        p = partial_for(xbuf[1], (my_pos + 2) % N_DEV)
        ca0.wait()
        asend[:, :] = (arecv[0] + p).astype(jnp.bfloat16)
        ca1 = acopy(1)
        ca1.start()

        cx2.wait()
        p = partial_for(xbuf[2], (my_pos + 1) % N_DEV)
        ca1.wait()
        asend[:, :] = (arecv[1] + p).astype(jnp.bfloat16)
        ca2 = acopy(2)
        ca2.start()

        p = partial_for(x_ref[0], my_pos)
        ca2.wait()
        out_ref[0, :, :] = arecv[2] + p

    out = pl.pallas_call(
        body,
        out_shape=jax.ShapeDtypeStruct((1, SQ, DM), jnp.float32),
        in_specs=[pl.BlockSpec(memory_space=pltpu.VMEM)] * 6,
        out_specs=pl.BlockSpec(memory_space=pltpu.VMEM),
        scratch_shapes=[
            pltpu.VMEM((3, SQ, DM), jnp.bfloat16),
            pltpu.VMEM((3, SQ, DM), jnp.bfloat16),
            pltpu.VMEM((SQ, DM), jnp.bfloat16),
            pltpu.SemaphoreType.DMA((3,)),
            pltpu.SemaphoreType.DMA((3,)),
            pltpu.SemaphoreType.DMA((3,)),
            pltpu.SemaphoreType.DMA((3,)),
        ],
        compiler_params=pltpu.CompilerParams(
            collective_id=0, vmem_limit_bytes=100 * 1024 * 1024,
        ),
    )(x_b, Wq_b, K_sl, V_sl, Wo_b, bias)
    return out


# device time: 222089 ns/iter; 1.1666x vs baseline; 1.1666x over previous
import numpy as np
import jax
import jax.numpy as jnp
from jax import lax
from jax.experimental import pallas as pl
from jax.experimental.pallas import tpu as pltpu

N_DEV = 4
SQ = 1024
SKV = 1024
H_LOC = 8
DH = 128
DM = 1024
SCALE = 0.08838834764831843

_qb = (np.arange(SQ) // 64)[:, None]
_kb = (np.arange(SKV) // 64)[None, :]
_mask = (_qb == _kb) | (_kb == 0) | ((_qb + _kb) % 3 == 0)
_BIAS = np.where(_mask, 0.0, -1e9).astype(np.float32)


def kernel(x, Wq, K_ext, V_ext, Wo):
    my = lax.axis_index("i")
    K_sl = lax.dynamic_slice_in_dim(K_ext, my * H_LOC, H_LOC, axis=2)
    V_sl = lax.dynamic_slice_in_dim(V_ext, my * H_LOC, H_LOC, axis=2)
    K_sl = K_sl.transpose(0, 2, 1, 3).astype(jnp.bfloat16)
    V_sl = V_sl.transpose(0, 2, 1, 3).astype(jnp.bfloat16)
    x_b = x.astype(jnp.bfloat16)
    Wq_b = Wq.astype(jnp.bfloat16).reshape(DM, H_LOC, DH).transpose(1, 0, 2)
    Wo_b = Wo.astype(jnp.bfloat16).reshape(H_LOC, DH, DM)
    bias = jnp.asarray(_BIAS, dtype=jnp.bfloat16)

    def body(x_ref, wq_ref, k_ref, v_ref, wo_ref, bias_ref, out_ref,
             xbuf, arecv, asend, xs, xr, as_, ar):
        my_pos = lax.axis_index("i")
        left = (my_pos + N_DEV - 1) % N_DEV
        right = (my_pos + 1) % N_DEV

        def partial_for(x_val, b):

            def head_body(h, acc):
                q = (lax.dot_general(
                    x_val, wq_ref[h], (((1,), (0,)), ((), ())),
                    preferred_element_type=jnp.float32,
                ) * SCALE).astype(jnp.bfloat16)
                s = lax.dot_general(
                    q, k_ref[b, h], (((1,), (1,)), ((), ())),
                    preferred_element_type=jnp.float32,
                )
                w = jnp.exp(s + bias_ref[:, :])
                denom = jnp.sum(w, axis=1, keepdims=True)
                ctx = lax.dot_general(
                    w.astype(jnp.bfloat16), v_ref[b, h],
                    (((1,), (0,)), ((), ())),
                    preferred_element_type=jnp.float32,
                ) / denom
                return acc + lax.dot_general(
                    ctx.astype(jnp.bfloat16), wo_ref[h],
                    (((1,), (0,)), ((), ())),
                    preferred_element_type=jnp.float32,
                )

            return lax.fori_loop(
                0, H_LOC, head_body, jnp.zeros((SQ, DM), jnp.float32)
            )

        def xcopy(src, slot):
            return pltpu.make_async_remote_copy(
                src_ref=src, dst_ref=xbuf.at[slot],
                send_sem=xs.at[slot], recv_sem=xr.at[slot],
                device_id=(right,), device_id_type=pl.DeviceIdType.MESH,
            )

        def acopy(slot):
            return pltpu.make_async_remote_copy(
                src_ref=asend, dst_ref=arecv.at[slot],
                send_sem=as_.at[slot], recv_sem=ar.at[slot],
                device_id=(right,), device_id_type=pl.DeviceIdType.MESH,
            )

        barrier = pltpu.get_barrier_semaphore()
        for nbr in (left, right):
            pl.semaphore_signal(
                barrier, inc=1,
                device_id=(nbr,), device_id_type=pl.DeviceIdType.MESH,
            )
        pl.semaphore_wait(barrier, 2)

        cx0 = xcopy(x_ref.at[0], 0)
        cx0.start()
        p_own = partial_for(x_ref[0], my_pos)

        cx0.wait()
        cx1 = xcopy(xbuf.at[0], 1)
        cx1.start()
        p = partial_for(xbuf[0], (my_pos + 3) % N_DEV)
        asend[:, :] = p.astype(jnp.bfloat16)
        ca0 = acopy(0)
        ca0.start()

        cx1.wait()
        cx2 = xcopy(xbuf.at[1], 2)
        cx2.start()
        p = partial_for(xbuf[1], (my_pos + 2) % N_DEV)
        ca0.wait()
        asend[:, :] = (arecv[0] + p).astype(jnp.bfloat16)
        ca1 = acopy(1)
        ca1.start()

        cx2.wait()
        p = partial_for(xbuf[2], (my_pos + 1) % N_DEV)
        ca1.wait()
        asend[:, :] = (arecv[1] + p).astype(jnp.bfloat16)
        ca2 = acopy(2)
        ca2.start()

        ca2.wait()
        out_ref[0, :, :] = arecv[2] + p_own

    out = pl.pallas_call(
        body,
        out_shape=jax.ShapeDtypeStruct((1, SQ, DM), jnp.float32),
        in_specs=[pl.BlockSpec(memory_space=pltpu.VMEM)] * 6,
        out_specs=pl.BlockSpec(memory_space=pltpu.VMEM),
        scratch_shapes=[
            pltpu.VMEM((3, SQ, DM), jnp.bfloat16),
            pltpu.VMEM((3, SQ, DM), jnp.bfloat16),
            pltpu.VMEM((SQ, DM), jnp.bfloat16),
            pltpu.SemaphoreType.DMA((3,)),
            pltpu.SemaphoreType.DMA((3,)),
            pltpu.SemaphoreType.DMA((3,)),
            pltpu.SemaphoreType.DMA((3,)),
        ],
        compiler_params=pltpu.CompilerParams(
            collective_id=0, vmem_limit_bytes=100 * 1024 * 1024,
        ),
    )(x_b, Wq_b, K_sl, V_sl, Wo_b, bias)
    return out
